# Initial kernel scaffold; baseline (speedup 1.0000x reference)
import functools

import jax
import jax.numpy as jnp
from jax import lax
from jax.experimental import pallas as pl
from jax.experimental.pallas import tpu as pltpu
from jax.experimental.pallas import tpu_sc as plsc

N_NODES = 10000
N_EDGES = 320000
D = 128
DH = 64

CHUNK = 128
NTILES = 16
NCORES = 2
NCHUNKS_PAD = 2512
CHUNKS_PER_TILE = NCHUNKS_PAD // NTILES
E_PAD = NCHUNKS_PAD * CHUNK
ACC_ROWS = 10240
ROWS_PER_TILE_Z = ACC_ROWS // NTILES
OUT_ROWS_PER_TILE = N_NODES // NTILES


def _mlp_body(x_ref, w_ref, b_ref, hlo_ref, hhi_ref):
    h = lax.dot_general(
        x_ref[...], w_ref[...], (((1,), (1,)), ((), ())),
        preferred_element_type=jnp.float32,
    ) + b_ref[...]
    hlo_ref[...] = h[:, :DH]
    hhi_ref[...] = h[:, DH:]


def _mlp(x, W, b):
    return pl.pallas_call(
        _mlp_body,
        grid=(10,),
        in_specs=[
            pl.BlockSpec((1000, D), lambda i: (i, 0)),
            pl.BlockSpec((D, D), lambda i: (0, 0)),
            pl.BlockSpec((1, D), lambda i: (0, 0)),
        ],
        out_specs=[
            pl.BlockSpec((1000, DH), lambda i: (i, 0)),
            pl.BlockSpec((1000, DH), lambda i: (i, 0)),
        ],
        out_shape=[
            jax.ShapeDtypeStruct((N_NODES, DH), jnp.float32),
            jax.ShapeDtypeStruct((N_NODES, DH), jnp.float32),
        ],
    )(x, W, b.reshape(1, D))


def _sc_body(col_hbm, row_hbm, hlo_hbm, hhi_hbm, out_hbm,
             col_v, row_v, gbuf, zbuf, acc, sem):
    cid = lax.axis_index("c")
    sid = lax.axis_index("s")
    base = sid * CHUNKS_PER_TILE

    pltpu.sync_copy(col_hbm.at[pl.ds(base, CHUNKS_PER_TILE)], col_v)
    pltpu.sync_copy(row_hbm.at[pl.ds(base, CHUNKS_PER_TILE)], row_v)

    zeros16 = jnp.zeros((16,), jnp.float32)

    def zb(i, carry):
        zbuf[i // 4, pl.ds((i % 4) * 16, 16)] = zeros16
        return carry

    lax.fori_loop(0, 512, zb, 0)
    for k in range(ROWS_PER_TILE_Z // CHUNK):
        pltpu.sync_copy(zbuf, acc.at[pl.ds(sid * ROWS_PER_TILE_Z + k * CHUNK, CHUNK)])
    plsc.subcore_barrier()

    def do_edges(h_hbm):
        def body(j, carry):
            pltpu.async_copy(h_hbm.at[col_v.at[j]], gbuf, sem).wait()
            pltpu.sync_copy(gbuf, acc.at[row_v.at[j]], add=True)
            return carry
        lax.fori_loop(0, CHUNKS_PER_TILE, body, 0)

    @pl.when(cid == 0)
    def _():
        do_edges(hlo_hbm)

    @pl.when(cid == 1)
    def _():
        do_edges(hhi_hbm)

    plsc.subcore_barrier()

    pltpu.sync_copy(
        acc.at[pl.ds(sid * OUT_ROWS_PER_TILE, OUT_ROWS_PER_TILE)],
        out_hbm.at[pl.ds(sid * OUT_ROWS_PER_TILE, OUT_ROWS_PER_TILE),
                   pl.ds(cid * DH, DH)],
    )


_sc_call = functools.partial(
    pl.kernel,
    mesh=plsc.VectorSubcoreMesh(
        core_axis_name="c", subcore_axis_name="s",
        num_cores=NCORES, num_subcores=NTILES,
    ),
    out_type=jax.ShapeDtypeStruct((N_NODES, D), jnp.float32),
    scratch_types=[
        pltpu.VMEM((CHUNKS_PER_TILE, CHUNK), jnp.int32),
        pltpu.VMEM((CHUNKS_PER_TILE, CHUNK), jnp.int32),
        pltpu.VMEM((CHUNK, DH), jnp.float32),
        pltpu.VMEM((CHUNK, DH), jnp.float32),
        pltpu.VMEM_SHARED((ACC_ROWS, DH), jnp.float32),
        pltpu.SemaphoreType.DMA,
    ],
)(_sc_body)


@jax.jit
def kernel(x, edge_index, W, b):
    row = edge_index[0].astype(jnp.int32)
    col = edge_index[1].astype(jnp.int32)
    pad = E_PAD - N_EDGES
    colp = jnp.concatenate([col, jnp.zeros((pad,), jnp.int32)]).reshape(
        NCHUNKS_PAD, CHUNK)
    rowp = jnp.concatenate([row, jnp.full((pad,), N_NODES, jnp.int32)]).reshape(
        NCHUNKS_PAD, CHUNK)
    h_lo, h_hi = _mlp(x, W, b)
    return _sc_call(colp, rowp, h_lo, h_hi)


# speedup vs baseline: 3.4935x; 3.4935x over previous
import functools

import jax
import jax.numpy as jnp
from jax import lax
from jax.experimental import pallas as pl
from jax.experimental.pallas import tpu as pltpu
from jax.experimental.pallas import tpu_sc as plsc

N_NODES = 10000
N_EDGES = 320000
D = 128

CHUNK = 128
NTILES = 16
NCORES = 2
NCHUNKS_PAD = 2560
CHUNKS_PER_CORE = NCHUNKS_PAD // NCORES
CHUNKS_PER_TILE = CHUNKS_PER_CORE // NTILES
E_PAD = NCHUNKS_PAD * CHUNK
ACC_ROWS = 10240
ROWS_PER_TILE = ACC_ROWS // NTILES
NIDX = 8


def _mlp_body(x_ref, w_ref, b_ref, h_ref):
    h_ref[...] = lax.dot_general(
        x_ref[...], w_ref[...], (((1,), (1,)), ((), ())),
        preferred_element_type=jnp.float32,
    ) + b_ref[...]


def _mlp(x, W, b):
    return pl.pallas_call(
        _mlp_body,
        grid=(10,),
        in_specs=[
            pl.BlockSpec((1000, D), lambda i: (i, 0)),
            pl.BlockSpec((D, D), lambda i: (0, 0)),
            pl.BlockSpec((1, D), lambda i: (0, 0)),
        ],
        out_specs=pl.BlockSpec((1000, D), lambda i: (i, 0)),
        out_shape=jax.ShapeDtypeStruct((N_NODES, D), jnp.float32),
    )(x, W, b.reshape(1, D))


def _add_body(p0_ref, p1_ref, o_ref):
    o_ref[...] = p0_ref[...] + p1_ref[...]


def _combine(p0, p1):
    return pl.pallas_call(
        _add_body,
        grid=(10,),
        in_specs=[
            pl.BlockSpec((1000, D), lambda i: (i, 0)),
            pl.BlockSpec((1000, D), lambda i: (i, 0)),
        ],
        out_specs=pl.BlockSpec((1000, D), lambda i: (i, 0)),
        out_shape=jax.ShapeDtypeStruct((N_NODES, D), jnp.float32),
    )(p0, p1)


def _sc_body(col_hbm, row_hbm, h_hbm, p0_hbm, p1_hbm,
             col_v, row_v, gbuf, acc, sem):
    cid = lax.axis_index("c")
    sid = lax.axis_index("s")
    base = cid * CHUNKS_PER_CORE + sid * CHUNKS_PER_TILE

    zeros16 = jnp.zeros((16,), jnp.float32)

    def zb(i, carry):
        gbuf[i // 8, pl.ds((i % 8) * 16, 16)] = zeros16
        return carry

    lax.fori_loop(0, CHUNK * 8, zb, 0)
    for k in range(ROWS_PER_TILE // CHUNK):
        pltpu.sync_copy(gbuf, acc.at[pl.ds(sid * ROWS_PER_TILE + k * CHUNK, CHUNK)])
    plsc.subcore_barrier()

    def outer(k, carry):
        pltpu.sync_copy(col_hbm.at[pl.ds(base + k * NIDX, NIDX)], col_v)
        pltpu.sync_copy(row_hbm.at[pl.ds(base + k * NIDX, NIDX)], row_v)

        def body(j, carry2):
            pltpu.async_copy(h_hbm.at[col_v.at[j]], gbuf, sem).wait()
            pltpu.sync_copy(gbuf, acc.at[row_v.at[j]], add=True)
            return carry2

        lax.fori_loop(0, NIDX, body, 0)
        return carry

    lax.fori_loop(0, CHUNKS_PER_TILE // NIDX, outer, 0)

    plsc.subcore_barrier()

    def writeout(p_hbm):
        pltpu.sync_copy(
            acc.at[pl.ds(sid * ROWS_PER_TILE, ROWS_PER_TILE)],
            p_hbm.at[pl.ds(sid * ROWS_PER_TILE, ROWS_PER_TILE)],
        )

    @pl.when(cid == 0)
    def _():
        writeout(p0_hbm)

    @pl.when(cid == 1)
    def _():
        writeout(p1_hbm)


_sc_call_cache = []


def _sc_call(*args):
    if not _sc_call_cache:
        _sc_call_cache.append(functools.partial(
            pl.kernel,
            mesh=plsc.VectorSubcoreMesh(
                core_axis_name="c", subcore_axis_name="s",
            ),
            out_type=[
                jax.ShapeDtypeStruct((ACC_ROWS, D), jnp.float32),
                jax.ShapeDtypeStruct((ACC_ROWS, D), jnp.float32),
            ],
            scratch_types=[
                pltpu.VMEM((NIDX, CHUNK), jnp.int32),
                pltpu.VMEM((NIDX, CHUNK), jnp.int32),
                pltpu.VMEM((CHUNK, D), jnp.float32),
                pltpu.VMEM_SHARED((ACC_ROWS, D), jnp.float32),
                pltpu.SemaphoreType.DMA,
            ],
        )(_sc_body))
    return _sc_call_cache[0](*args)


@jax.jit
def kernel(x, edge_index, W, b):
    row = edge_index[0].astype(jnp.int32)
    col = edge_index[1].astype(jnp.int32)
    pad = E_PAD - N_EDGES
    colp = jnp.concatenate([col, jnp.zeros((pad,), jnp.int32)]).reshape(
        NCHUNKS_PAD, CHUNK)
    rowp = jnp.concatenate([row, jnp.full((pad,), N_NODES, jnp.int32)]).reshape(
        NCHUNKS_PAD, CHUNK)
    h = _mlp(x, W, b)
    p0, p1 = _sc_call(colp, rowp, h)
    return _combine(p0, p1)


# speedup vs baseline: 4.0222x; 1.1513x over previous
import functools

import jax
import jax.numpy as jnp
from jax import lax
from jax.experimental import pallas as pl
from jax.experimental.pallas import tpu as pltpu
from jax.experimental.pallas import tpu_sc as plsc

N_NODES = 10000
N_EDGES = 320000
D = 128

CHUNK = 128
NTILES = 16
NCORES = 2
NCHUNKS_PAD = 2560
CHUNKS_PER_CORE = NCHUNKS_PAD // NCORES
CHUNKS_PER_TILE = CHUNKS_PER_CORE // NTILES
E_PAD = NCHUNKS_PAD * CHUNK
ACC_ROWS = 10240
ROWS_PER_TILE = ACC_ROWS // NTILES
NIDX = 40


def _mlp_body(x_ref, w_ref, b_ref, h_ref):
    h_ref[...] = lax.dot_general(
        x_ref[...], w_ref[...], (((1,), (1,)), ((), ())),
        preferred_element_type=jnp.float32,
    ) + b_ref[...]


def _mlp(x, W, b):
    return pl.pallas_call(
        _mlp_body,
        grid=(10,),
        in_specs=[
            pl.BlockSpec((1000, D), lambda i: (i, 0)),
            pl.BlockSpec((D, D), lambda i: (0, 0)),
            pl.BlockSpec((1, D), lambda i: (0, 0)),
        ],
        out_specs=pl.BlockSpec((1000, D), lambda i: (i, 0)),
        out_shape=jax.ShapeDtypeStruct((N_NODES, D), jnp.float32),
    )(x, W, b.reshape(1, D))


def _add_body(p0_ref, p1_ref, o_ref):
    o_ref[...] = p0_ref[...] + p1_ref[...]


def _combine(p0, p1):
    return pl.pallas_call(
        _add_body,
        grid=(10,),
        in_specs=[
            pl.BlockSpec((1000, D), lambda i: (i, 0)),
            pl.BlockSpec((1000, D), lambda i: (i, 0)),
        ],
        out_specs=pl.BlockSpec((1000, D), lambda i: (i, 0)),
        out_shape=jax.ShapeDtypeStruct((N_NODES, D), jnp.float32),
    )(p0, p1)


def _sc_body(col_hbm, row_hbm, h_hbm, p0_hbm, p1_hbm,
             col_v, row_v, gbuf0, gbuf1, acc, sem0, sem1):
    cid = lax.axis_index("c")
    sid = lax.axis_index("s")
    base = cid * CHUNKS_PER_CORE + sid * CHUNKS_PER_TILE

    zeros16 = jnp.zeros((16,), jnp.float32)

    def zb(i, carry):
        gbuf0[i // 8, pl.ds((i % 8) * 16, 16)] = zeros16
        return carry

    lax.fori_loop(0, CHUNK * 8, zb, 0)
    for k in range(ROWS_PER_TILE // CHUNK):
        pltpu.sync_copy(gbuf0, acc.at[pl.ds(sid * ROWS_PER_TILE + k * CHUNK, CHUNK)])
    plsc.subcore_barrier()

    for k in range(CHUNKS_PER_TILE // NIDX):
        pltpu.sync_copy(col_hbm.at[pl.ds(base + k * NIDX, NIDX)], col_v)
        pltpu.sync_copy(row_hbm.at[pl.ds(base + k * NIDX, NIDX)], row_v)

        pltpu.async_copy(h_hbm.at[col_v.at[0]], gbuf0, sem0)

        def pair(m, carry2):
            pltpu.async_copy(h_hbm.at[col_v.at[2 * m + 1]], gbuf1, sem1)
            pltpu.make_async_copy(h_hbm.at[col_v.at[2 * m]], gbuf0, sem0).wait()
            pltpu.sync_copy(gbuf0, acc.at[row_v.at[2 * m]], add=True)

            @pl.when(m < NIDX // 2 - 1)
            def _():
                pltpu.async_copy(h_hbm.at[col_v.at[2 * m + 2]], gbuf0, sem0)

            pltpu.make_async_copy(
                h_hbm.at[col_v.at[2 * m + 1]], gbuf1, sem1).wait()
            pltpu.sync_copy(gbuf1, acc.at[row_v.at[2 * m + 1]], add=True)
            return carry2

        lax.fori_loop(0, NIDX // 2, pair, 0)

    plsc.subcore_barrier()

    def writeout(p_hbm):
        pltpu.sync_copy(
            acc.at[pl.ds(sid * ROWS_PER_TILE, ROWS_PER_TILE)],
            p_hbm.at[pl.ds(sid * ROWS_PER_TILE, ROWS_PER_TILE)],
        )

    @pl.when(cid == 0)
    def _():
        writeout(p0_hbm)

    @pl.when(cid == 1)
    def _():
        writeout(p1_hbm)


_sc_call_cache = []


def _sc_call(*args):
    if not _sc_call_cache:
        _sc_call_cache.append(functools.partial(
            pl.kernel,
            mesh=plsc.VectorSubcoreMesh(
                core_axis_name="c", subcore_axis_name="s",
            ),
            out_type=[
                jax.ShapeDtypeStruct((ACC_ROWS, D), jnp.float32),
                jax.ShapeDtypeStruct((ACC_ROWS, D), jnp.float32),
            ],
            scratch_types=[
                pltpu.VMEM((NIDX, CHUNK), jnp.int32),
                pltpu.VMEM((NIDX, CHUNK), jnp.int32),
                pltpu.VMEM((CHUNK, D), jnp.float32),
                pltpu.VMEM((CHUNK, D), jnp.float32),
                pltpu.VMEM_SHARED((ACC_ROWS, D), jnp.float32),
                pltpu.SemaphoreType.DMA,
                pltpu.SemaphoreType.DMA,
            ],
        )(_sc_body))
    return _sc_call_cache[0](*args)


@jax.jit
def kernel(x, edge_index, W, b):
    row = edge_index[0].astype(jnp.int32)
    col = edge_index[1].astype(jnp.int32)
    pad = E_PAD - N_EDGES
    colp = jnp.concatenate([col, jnp.zeros((pad,), jnp.int32)]).reshape(
        NCHUNKS_PAD, CHUNK)
    rowp = jnp.concatenate([row, jnp.full((pad,), N_NODES, jnp.int32)]).reshape(
        NCHUNKS_PAD, CHUNK)
    h = _mlp(x, W, b)
    p0, p1 = _sc_call(colp, rowp, h)
    return _combine(p0, p1)


# speedup vs baseline: 4.0229x; 1.0002x over previous
import functools

import jax
import jax.numpy as jnp
from jax import lax
from jax.experimental import pallas as pl
from jax.experimental.pallas import tpu as pltpu
from jax.experimental.pallas import tpu_sc as plsc

N_NODES = 10000
N_EDGES = 320000
D = 128

CHUNK = 128
NTILES = 16
NCORES = 2
NCHUNKS_PAD = 2560
CHUNKS_PER_CORE = NCHUNKS_PAD // NCORES
CHUNKS_PER_TILE = CHUNKS_PER_CORE // NTILES
E_PAD = NCHUNKS_PAD * CHUNK
ACC_ROWS = 10240
ROWS_PER_TILE = ACC_ROWS // NTILES
NIDX = 40


def _mlp_body(x_ref, w_ref, b_ref, h_ref):
    h_ref[...] = lax.dot_general(
        x_ref[...], w_ref[...], (((1,), (1,)), ((), ())),
        preferred_element_type=jnp.float32,
    ) + b_ref[...]


def _mlp(x, W, b):
    return pl.pallas_call(
        _mlp_body,
        grid=(10,),
        in_specs=[
            pl.BlockSpec((1000, D), lambda i: (i, 0)),
            pl.BlockSpec((D, D), lambda i: (0, 0)),
            pl.BlockSpec((1, D), lambda i: (0, 0)),
        ],
        out_specs=pl.BlockSpec((1000, D), lambda i: (i, 0)),
        out_shape=jax.ShapeDtypeStruct((N_NODES, D), jnp.float32),
    )(x, W, b.reshape(1, D))


def _add_body(p0_ref, p1_ref, o_ref):
    o_ref[...] = p0_ref[...] + p1_ref[...]


def _combine(p0, p1):
    return pl.pallas_call(
        _add_body,
        grid=(10,),
        in_specs=[
            pl.BlockSpec((1000, D), lambda i: (i, 0)),
            pl.BlockSpec((1000, D), lambda i: (i, 0)),
        ],
        out_specs=pl.BlockSpec((1000, D), lambda i: (i, 0)),
        out_shape=jax.ShapeDtypeStruct((N_NODES, D), jnp.float32),
    )(p0, p1)


def _sc_body(col_hbm, row_hbm, h_hbm, p0_hbm, p1_hbm,
             col_v, row_v, gbuf0, gbuf1, acc, sem0, sem1):
    cid = lax.axis_index("c")
    sid = lax.axis_index("s")
    base = cid * CHUNKS_PER_CORE + sid * CHUNKS_PER_TILE

    zeros16 = jnp.zeros((16,), jnp.float32)

    def zb(i, carry):
        gbuf0[i // 8, pl.ds((i % 8) * 16, 16)] = zeros16
        return carry

    with jax.named_scope("zero_acc"):
        lax.fori_loop(0, CHUNK * 8, zb, 0)
        for k in range(ROWS_PER_TILE // CHUNK):
            pltpu.sync_copy(
                gbuf0, acc.at[pl.ds(sid * ROWS_PER_TILE + k * CHUNK, CHUNK)])
        plsc.subcore_barrier()

    for k in range(CHUNKS_PER_TILE // NIDX):
      with jax.named_scope(f"edges_blk{k}"):
        pltpu.sync_copy(col_hbm.at[pl.ds(base + k * NIDX, NIDX)], col_v)
        pltpu.sync_copy(row_hbm.at[pl.ds(base + k * NIDX, NIDX)], row_v)

        pltpu.async_copy(h_hbm.at[col_v.at[0]], gbuf0, sem0)

        def pair(m, carry2):
            pltpu.async_copy(h_hbm.at[col_v.at[2 * m + 1]], gbuf1, sem1)
            pltpu.make_async_copy(h_hbm.at[col_v.at[2 * m]], gbuf0, sem0).wait()
            pltpu.sync_copy(gbuf0, acc.at[row_v.at[2 * m]], add=True)

            @pl.when(m < NIDX // 2 - 1)
            def _():
                pltpu.async_copy(h_hbm.at[col_v.at[2 * m + 2]], gbuf0, sem0)

            pltpu.make_async_copy(
                h_hbm.at[col_v.at[2 * m + 1]], gbuf1, sem1).wait()
            pltpu.sync_copy(gbuf1, acc.at[row_v.at[2 * m + 1]], add=True)
            return carry2

        lax.fori_loop(0, NIDX // 2, pair, 0)

    with jax.named_scope("post_barrier"):
        plsc.subcore_barrier()

    def writeout(p_hbm):
        pltpu.sync_copy(
            acc.at[pl.ds(sid * ROWS_PER_TILE, ROWS_PER_TILE)],
            p_hbm.at[pl.ds(sid * ROWS_PER_TILE, ROWS_PER_TILE)],
        )

    @pl.when(cid == 0)
    def _():
        writeout(p0_hbm)

    @pl.when(cid == 1)
    def _():
        writeout(p1_hbm)


_sc_call_cache = []


def _sc_call(*args):
    if not _sc_call_cache:
        _sc_call_cache.append(functools.partial(
            pl.kernel,
            mesh=plsc.VectorSubcoreMesh(
                core_axis_name="c", subcore_axis_name="s",
            ),
            out_type=[
                jax.ShapeDtypeStruct((ACC_ROWS, D), jnp.float32),
                jax.ShapeDtypeStruct((ACC_ROWS, D), jnp.float32),
            ],
            scratch_types=[
                pltpu.VMEM((NIDX, CHUNK), jnp.int32),
                pltpu.VMEM((NIDX, CHUNK), jnp.int32),
                pltpu.VMEM((CHUNK, D), jnp.float32),
                pltpu.VMEM((CHUNK, D), jnp.float32),
                pltpu.VMEM_SHARED((ACC_ROWS, D), jnp.float32),
                pltpu.SemaphoreType.DMA,
                pltpu.SemaphoreType.DMA,
            ],
        )(_sc_body))
    return _sc_call_cache[0](*args)


@jax.jit
def kernel(x, edge_index, W, b):
    row = edge_index[0].astype(jnp.int32)
    col = edge_index[1].astype(jnp.int32)
    pad = E_PAD - N_EDGES
    colp = jnp.concatenate([col, jnp.zeros((pad,), jnp.int32)]).reshape(
        NCHUNKS_PAD, CHUNK)
    rowp = jnp.concatenate([row, jnp.full((pad,), N_NODES, jnp.int32)]).reshape(
        NCHUNKS_PAD, CHUNK)
    h = _mlp(x, W, b)
    p0, p1 = _sc_call(colp, rowp, h)
    return _combine(p0, p1)


# speedup vs baseline: 11.4715x; 2.8516x over previous
import functools

import jax
import jax.numpy as jnp
import numpy as np
from jax import lax
from jax.experimental import pallas as pl
from jax.experimental.pallas import tpu as pltpu
from jax.experimental.pallas import tpu_sc as plsc

N_NODES = 10000
N_EDGES = 320000
D = 128

CHUNK = 128
NTILES = 16
NCORES = 2
NCHUNKS_PAD = 2560
CHUNKS_PER_CORE = NCHUNKS_PAD // NCORES
CHUNKS_PER_TILE = CHUNKS_PER_CORE // NTILES
E_PAD = NCHUNKS_PAD * CHUNK
ACC_ROWS = 10240
ROWS_PER_TILE = ACC_ROWS // NTILES
NIDX = 40

N_REAL_CHUNKS = N_EDGES // CHUNK
N_PAD_CHUNKS = NCHUNKS_PAD - N_REAL_CHUNKS

def _make_slot_src():
    nslots = CHUNKS_PER_TILE
    ntiles_all = NCORES * NTILES
    base_real = N_REAL_CHUNKS // ntiles_all
    extra = N_REAL_CHUNKS - base_real * ntiles_all
    slot_src = np.empty((NCHUNKS_PAD,), np.int32)
    rc, pc = 0, N_REAL_CHUNKS
    for t in range(ntiles_all):
        n_real = base_real + (1 if t < extra else 0)
        for j in range(nslots):
            if j < n_real:
                slot_src[t * nslots + j] = rc
                rc += 1
            else:
                slot_src[t * nslots + j] = pc
                pc += 1
    assert rc == N_REAL_CHUNKS and pc == NCHUNKS_PAD
    return slot_src

_SLOT_SRC = _make_slot_src()
_PAD_COL = np.tile(np.arange(CHUNK, dtype=np.int32), (N_PAD_CHUNKS, 1))
_PAD_ROW = N_NODES + _PAD_COL


def _mlp_body(x_ref, w_ref, b_ref, h_ref):
    h_ref[...] = lax.dot_general(
        x_ref[...], w_ref[...], (((1,), (1,)), ((), ())),
        preferred_element_type=jnp.float32,
    ) + b_ref[...]


def _mlp(x, W, b):
    return pl.pallas_call(
        _mlp_body,
        grid=(10,),
        in_specs=[
            pl.BlockSpec((1000, D), lambda i: (i, 0)),
            pl.BlockSpec((D, D), lambda i: (0, 0)),
            pl.BlockSpec((1, D), lambda i: (0, 0)),
        ],
        out_specs=pl.BlockSpec((1000, D), lambda i: (i, 0)),
        out_shape=jax.ShapeDtypeStruct((N_NODES, D), jnp.float32),
    )(x, W, b.reshape(1, D))


def _add_body(p0_ref, p1_ref, o_ref):
    o_ref[...] = p0_ref[...] + p1_ref[...]


def _combine(p0, p1):
    return pl.pallas_call(
        _add_body,
        grid=(10,),
        in_specs=[
            pl.BlockSpec((1000, D), lambda i: (i, 0)),
            pl.BlockSpec((1000, D), lambda i: (i, 0)),
        ],
        out_specs=pl.BlockSpec((1000, D), lambda i: (i, 0)),
        out_shape=jax.ShapeDtypeStruct((N_NODES, D), jnp.float32),
    )(p0, p1)


def _sc_body(col_hbm, row_hbm, h_hbm, p0_hbm, p1_hbm,
             col_v, row_v, gbuf0, gbuf1, acc, sem0, sem1):
    cid = lax.axis_index("c")
    sid = lax.axis_index("s")
    base = cid * CHUNKS_PER_CORE + sid * CHUNKS_PER_TILE

    zeros16 = jnp.zeros((16,), jnp.float32)

    def zb(i, carry):
        gbuf0[i // 8, pl.ds((i % 8) * 16, 16)] = zeros16
        return carry

    with jax.named_scope("zero_acc"):
        lax.fori_loop(0, CHUNK * 8, zb, 0)
        for k in range(ROWS_PER_TILE // CHUNK):
            pltpu.sync_copy(
                gbuf0, acc.at[pl.ds(sid * ROWS_PER_TILE + k * CHUNK, CHUNK)])
        plsc.subcore_barrier()

    for k in range(CHUNKS_PER_TILE // NIDX):
      with jax.named_scope(f"edges_blk{k}"):
        pltpu.sync_copy(col_hbm.at[pl.ds(base + k * NIDX, NIDX)], col_v)
        pltpu.sync_copy(row_hbm.at[pl.ds(base + k * NIDX, NIDX)], row_v)

        pltpu.async_copy(h_hbm.at[col_v.at[0]], gbuf0, sem0)

        def pair(m, carry2):
            pltpu.async_copy(h_hbm.at[col_v.at[2 * m + 1]], gbuf1, sem1)
            pltpu.make_async_copy(h_hbm.at[col_v.at[2 * m]], gbuf0, sem0).wait()
            pltpu.sync_copy(gbuf0, acc.at[row_v.at[2 * m]], add=True)

            @pl.when(m < NIDX // 2 - 1)
            def _():
                pltpu.async_copy(h_hbm.at[col_v.at[2 * m + 2]], gbuf0, sem0)

            pltpu.make_async_copy(
                h_hbm.at[col_v.at[2 * m + 1]], gbuf1, sem1).wait()
            pltpu.sync_copy(gbuf1, acc.at[row_v.at[2 * m + 1]], add=True)
            return carry2

        lax.fori_loop(0, NIDX // 2, pair, 0)

    with jax.named_scope("post_barrier"):
        plsc.subcore_barrier()

    def writeout(p_hbm):
        pltpu.sync_copy(
            acc.at[pl.ds(sid * ROWS_PER_TILE, ROWS_PER_TILE)],
            p_hbm.at[pl.ds(sid * ROWS_PER_TILE, ROWS_PER_TILE)],
        )

    @pl.when(cid == 0)
    def _():
        writeout(p0_hbm)

    @pl.when(cid == 1)
    def _():
        writeout(p1_hbm)


_sc_call_cache = []


def _sc_call(*args):
    if not _sc_call_cache:
        _sc_call_cache.append(functools.partial(
            pl.kernel,
            mesh=plsc.VectorSubcoreMesh(
                core_axis_name="c", subcore_axis_name="s",
            ),
            out_type=[
                jax.ShapeDtypeStruct((ACC_ROWS, D), jnp.float32),
                jax.ShapeDtypeStruct((ACC_ROWS, D), jnp.float32),
            ],
            scratch_types=[
                pltpu.VMEM((NIDX, CHUNK), jnp.int32),
                pltpu.VMEM((NIDX, CHUNK), jnp.int32),
                pltpu.VMEM((CHUNK, D), jnp.float32),
                pltpu.VMEM((CHUNK, D), jnp.float32),
                pltpu.VMEM_SHARED((ACC_ROWS, D), jnp.float32),
                pltpu.SemaphoreType.DMA,
                pltpu.SemaphoreType.DMA,
            ],
        )(_sc_body))
    return _sc_call_cache[0](*args)


@jax.jit
def kernel(x, edge_index, W, b):
    row = edge_index[0].astype(jnp.int32)
    col = edge_index[1].astype(jnp.int32)
    colp = jnp.concatenate(
        [col.reshape(N_REAL_CHUNKS, CHUNK), jnp.asarray(_PAD_COL)]
    )[_SLOT_SRC]
    rowp = jnp.concatenate(
        [row.reshape(N_REAL_CHUNKS, CHUNK), jnp.asarray(_PAD_ROW)]
    )[_SLOT_SRC]
    h = _mlp(x, W, b)
    p0, p1 = _sc_call(colp, rowp, h)
    return _combine(p0, p1)
